# Initial kernel scaffold; baseline (speedup 1.0000x reference)
import jax
import jax.numpy as jnp
from jax.experimental import pallas as pl


def kernel(static, dynamic, x0, se_w, se_b, de_w, de_b, dec_w, dec_b, gru_wi, gru_wh, gru_bi, gru_bh, att_v, att_W, ptr_v, ptr_W):
    raise NotImplementedError("write your pallas kernel here")


# speedup vs baseline: 5.4595x; 5.4595x over previous
import jax
import jax.numpy as jnp
from jax.experimental import pallas as pl
from jax.experimental.pallas import tpu as pltpu

_B, _S, _H = 128, 50, 128
_NCHUNK = 16
_CB = _B // _NCHUNK


def _bf(x):
    return x.astype(jnp.bfloat16)


def _bfr(x):
    return x.astype(jnp.bfloat16).astype(jnp.float32)


def _sigmoid(x):
    return 1.0 / (1.0 + jnp.exp(-x))


def _decode_body(st0_ref, st1_ref, dyn0_ref,
                 sewT_ref, seb_ref, dewT_ref, deb_ref,
                 decT_ref, decb_ref, di0i_ref, di1i_ref,
                 wiT_ref, whT_ref, bi_ref, bh_ref,
                 w1T_ref, w2T_ref, w3T_ref, p1T_ref, p2T_ref,
                 vatt_ref, vptr_ref,
                 idx_out_ref, logp_out_ref,
                 sh2_ref, ea2_ref, ps2_ref):
    B, S, H = _B, _S, _H
    f32 = jnp.float32

    def pre(c, _):
        base = c * _CB
        s0 = st0_ref[pl.ds(base, _CB), :]
        s1 = st1_ref[pl.ds(base, _CB), :]
        d0 = dyn0_ref[pl.ds(base, _CB), :]
        sew = sewT_ref[...]
        dew = dewT_ref[...]
        seb = seb_ref[...]
        deb = deb_ref[...]
        sewr = _bfr(sew)
        sh = (_bfr(s0)[:, :, None] * sewr[0:1, None, :]
              + _bfr(s1)[:, :, None] * sewr[1:2, None, :]
              + seb[0:1, None, :])
        dh_bf = _bf(d0[:, :, None] * dew[0:1, None, :] + deb[0:1, None, :])
        sh_bf = _bf(sh)
        W1 = w1T_ref[...]
        W2 = w2T_ref[...]
        P1 = p1T_ref[...]
        for i in range(_CB):
            shi = sh_bf[i]
            dhi = dh_bf[i]
            ea = (jnp.dot(shi, W1, preferred_element_type=f32)
                  + jnp.dot(dhi, W2, preferred_element_type=f32))
            ps = jnp.dot(shi, P1, preferred_element_type=f32)
            sh2_ref[pl.ds(base + i, 1)] = sh[i][None]
            ea2_ref[pl.ds(base + i, 1)] = ea[None]
            ps2_ref[pl.ds(base + i, 1)] = ps[None]
        return 0

    jax.lax.fori_loop(0, _NCHUNK, pre, 0)

    iota_s = jax.lax.broadcasted_iota(jnp.int32, (B, S), 1)
    st0 = st0_ref[...]
    st1 = st1_ref[...]

    def step(t, carry):
        h, di0, di1, idx_acc, logp_acc = carry
        decT = decT_ref[...].astype(f32)
        dh = (_bfr(di0) * decT[0:1, :] + _bfr(di1) * decT[1:2, :]
              + decb_ref[...])
        gi = jnp.dot(_bf(dh), wiT_ref[...], preferred_element_type=f32) + bi_ref[...]
        gh = jnp.dot(_bf(h), whT_ref[...], preferred_element_type=f32) + bh_ref[...]
        r = _sigmoid(gi[:, :H] + gh[:, :H])
        z = _sigmoid(gi[:, H:2 * H] + gh[:, H:2 * H])
        n = jnp.tanh(gi[:, 2 * H:] + r * gh[:, 2 * H:])
        h = (1.0 - z) * n + z * h

        w3h = jnp.dot(_bf(h), w3T_ref[...], preferred_element_type=f32)
        vatt = vatt_ref[...].astype(f32)[:, None, :]
        parts = []
        for c in range(_NCHUNK):
            eac = ea2_ref[pl.ds(c * _CB, _CB)]
            wc = w3h[c * _CB:(c + 1) * _CB]
            e = _bfr(jnp.tanh(eac + wc[:, None, :]))
            parts.append(jnp.sum(e * vatt, axis=2))
        scores = jnp.concatenate(parts, axis=0)
        m = jnp.max(scores, axis=1, keepdims=True)
        ex = jnp.exp(scores - m)
        attns = ex / jnp.sum(ex, axis=1, keepdims=True)

        parts = []
        for c in range(_NCHUNK):
            shc = sh2_ref[pl.ds(c * _CB, _CB)]
            ac = attns[c * _CB:(c + 1) * _CB][:, :, None]
            parts.append(jnp.sum(shc * ac, axis=1))
        ctx = jnp.concatenate(parts, axis=0)

        p2c = jnp.dot(_bf(ctx), p2T_ref[...], preferred_element_type=f32)
        vptr = vptr_ref[...].astype(f32)[:, None, :]
        parts = []
        for c in range(_NCHUNK):
            psc = ps2_ref[pl.ds(c * _CB, _CB)]
            pc = p2c[c * _CB:(c + 1) * _CB]
            e2 = _bfr(jnp.tanh(psc + pc[:, None, :]))
            parts.append(jnp.sum(e2 * vptr, axis=2))
        logits = jnp.concatenate(parts, axis=0)

        m2 = jnp.max(logits, axis=1, keepdims=True)
        ex2 = jnp.exp(logits - m2)
        ssum = jnp.sum(ex2, axis=1, keepdims=True)
        probs = ex2 / ssum
        mp = jnp.max(probs, axis=1, keepdims=True)
        ptr = jnp.min(jnp.where(probs == mp, iota_s, S), axis=1,
                      keepdims=True)
        onehot = (iota_s == ptr).astype(f32)
        p_sel = jnp.sum(probs * onehot, axis=1, keepdims=True)
        logp_t = jnp.log(p_sel)
        di0 = jnp.sum(st0 * onehot, axis=1, keepdims=True)
        di1 = jnp.sum(st1 * onehot, axis=1, keepdims=True)
        lane_t = iota_s == t
        idx_acc = jnp.where(lane_t, jnp.broadcast_to(ptr, (B, S)), idx_acc)
        logp_acc = jnp.where(lane_t, jnp.broadcast_to(logp_t, (B, S)), logp_acc)
        return h, di0, di1, idx_acc, logp_acc

    carry0 = (jnp.zeros((B, H), f32),
              di0i_ref[...], di1i_ref[...],
              jnp.zeros((B, S), jnp.int32),
              jnp.zeros((B, S), f32))
    _, _, _, idx_acc, logp_acc = jax.lax.fori_loop(0, S, step, carry0)
    idx_out_ref[...] = idx_acc
    logp_out_ref[...] = logp_acc


def kernel(static, dynamic, x0, se_w, se_b, de_w, de_b, dec_w, dec_b,
           gru_wi, gru_wh, gru_bi, gru_bh, att_v, att_W, ptr_v, ptr_W):
    B, _, S = static.shape
    H = se_w.shape[0]
    bf16 = jnp.bfloat16
    st0 = static[:, 0, :]
    st1 = static[:, 1, :]
    dyn0 = dynamic[:, 0, :]
    A = att_W[0]
    W1T = A[:, :H].T.astype(bf16)
    W2T = A[:, H:2 * H].T.astype(bf16)
    W3T = A[:, 2 * H:].T.astype(bf16)
    Pw = ptr_W[0]
    P1T = Pw[:, :H].T.astype(bf16)
    P2T = Pw[:, H:].T.astype(bf16)
    di0i = jnp.broadcast_to(x0[0, 0, 0], (B, 1)).astype(jnp.float32)
    di1i = jnp.broadcast_to(x0[0, 1, 0], (B, 1)).astype(jnp.float32)
    idx, logp = pl.pallas_call(
        _decode_body,
        out_shape=(jax.ShapeDtypeStruct((B, S), jnp.int32),
                   jax.ShapeDtypeStruct((B, S), jnp.float32)),
        scratch_shapes=[pltpu.VMEM((B, S, H), jnp.float32),
                        pltpu.VMEM((B, S, H), jnp.float32),
                        pltpu.VMEM((B, S, H), jnp.float32)],
    )(st0, st1, dyn0,
      se_w.T, se_b[None, :], de_w.T, de_b[None, :],
      dec_w.T.astype(bf16), dec_b[None, :], di0i, di1i,
      gru_wi.T.astype(bf16), gru_wh.T.astype(bf16),
      gru_bi[None, :], gru_bh[None, :],
      W1T, W2T, W3T, P1T, P2T,
      att_v[0].astype(bf16), ptr_v[0].astype(bf16))
    return idx, logp
